# Initial kernel scaffold; baseline (speedup 1.0000x reference)
import jax
import jax.numpy as jnp
from jax.experimental import pallas as pl


def kernel(x, gate_w, gate_b, expert_w, expert_b):
    raise NotImplementedError("write your pallas kernel here")


# speedup vs baseline: 2.1282x; 2.1282x over previous
import jax
import jax.numpy as jnp
from jax.experimental import pallas as pl
from jax.experimental.pallas import tpu as pltpu

_TOKEN_BLOCK = 2048


def _moe_body(x_ref, gw_ref, gb_ref, ew_ref, eb_ref, o_ref, w_ref):
    e = pl.program_id(1)

    @pl.when(e == 0)
    def _gate():
        logits = jnp.dot(x_ref[...], gw_ref[...],
                         preferred_element_type=jnp.float32) + gb_ref[...]
        m = jnp.max(logits, axis=1, keepdims=True)
        p = jnp.exp(logits - m)
        w_ref[...] = p / jnp.sum(p, axis=1, keepdims=True)
        o_ref[...] = jnp.zeros_like(o_ref)

    lane = jax.lax.broadcasted_iota(jnp.int32, (1, w_ref.shape[1]), 1)
    w_e = jnp.sum(jnp.where(lane == e, w_ref[...], 0.0), axis=1, keepdims=True)

    xs = (x_ref[...] * w_e).astype(jnp.bfloat16)
    acc = jnp.dot(xs, ew_ref[0], preferred_element_type=jnp.float32)
    o_ref[...] += acc + w_e * eb_ref[0]


def kernel(x, gate_w, gate_b, expert_w, expert_b):
    tokens, f_in = x.shape
    n_exp, _, f_out = expert_w.shape
    bt = min(_TOKEN_BLOCK, tokens)
    grid = (tokens // bt, n_exp)

    return pl.pallas_call(
        _moe_body,
        grid=grid,
        in_specs=[
            pl.BlockSpec((bt, f_in), lambda i, e: (i, 0)),
            pl.BlockSpec((f_in, n_exp), lambda i, e: (0, 0)),
            pl.BlockSpec((1, n_exp), lambda i, e: (0, 0)),
            pl.BlockSpec((1, f_in, f_out), lambda i, e: (e, 0, 0)),
            pl.BlockSpec((1, 1, f_out), lambda i, e: (e, 0, 0)),
        ],
        out_specs=pl.BlockSpec((bt, f_out), lambda i, e: (i, 0)),
        out_shape=jax.ShapeDtypeStruct((tokens, f_out), jnp.float32),
        scratch_shapes=[pltpu.VMEM((bt, n_exp), jnp.float32)],
        compiler_params=pltpu.CompilerParams(
            dimension_semantics=("parallel", "arbitrary")),
    )(x, gate_w, gate_b.reshape(1, n_exp), expert_w.astype(jnp.bfloat16),
      expert_b.reshape(n_exp, 1, f_out))
